# baseline (speedup 1.0000x reference)
import functools

import jax
import jax.numpy as jnp
from jax import lax
from jax.experimental import pallas as pl
from jax.experimental.pallas import tpu as pltpu
from jax.experimental.pallas import tpu_sc as plsc

NUM_FIELDS = 8
VOCAB = 100000
EMB = 64
PROJ = 128

NC, NS = 2, 16
NW = NC * NS
CH = 128
GRP = 2
BLK = CH * GRP
NBUF = 4


def _sc_gather(flat_table, idx2d, total_rows):
    rpw = total_rows // NW
    nblk = rpw // BLK
    nround = nblk // NBUF

    mesh = plsc.VectorSubcoreMesh(core_axis_name="c", subcore_axis_name="s")

    @functools.partial(
        pl.kernel,
        out_type=jax.ShapeDtypeStruct((total_rows, EMB), jnp.float32),
        mesh=mesh,
        scratch_types=[
            pltpu.VMEM((NBUF, GRP, CH), jnp.int32),
            pltpu.VMEM((NBUF, BLK, EMB), jnp.float32),
        ]
        + [pltpu.SemaphoreType.DMA] * (2 * NBUF),
        compiler_params=pltpu.CompilerParams(use_tc_tiling_on_sc=False),
    )
    def k(tab_hbm, idx_hbm, out_hbm, idx_v, rows_v, *sems):
        gsem, osem = sems[:NBUF], sems[NBUF:]
        wid = lax.axis_index("s") * NC + lax.axis_index("c")
        row_base = wid * rpw
        idx_base = wid * (rpw // CH)

        def stage_and_fire(h, blk_i):
            pltpu.sync_copy(
                idx_hbm.at[pl.ds(idx_base + blk_i * GRP, GRP)], idx_v.at[h]
            )
            for j in range(GRP):
                pltpu.async_copy(
                    tab_hbm.at[idx_v.at[h].at[j]],
                    rows_v.at[h].at[pl.ds(j * CH, CH)],
                    gsem[h],
                )

        def wait_gather(h):
            for j in range(GRP):
                pltpu.make_async_copy(
                    tab_hbm.at[idx_v.at[h].at[j]],
                    rows_v.at[h].at[pl.ds(j * CH, CH)],
                    gsem[h],
                ).wait()

        def fire_out(h, blk_i):
            pltpu.async_copy(
                rows_v.at[h],
                out_hbm.at[pl.ds(row_base + blk_i * BLK, BLK)],
                osem[h],
            )

        def wait_out(h):
            pltpu.make_async_copy(
                rows_v.at[h], out_hbm.at[pl.ds(row_base, BLK)], osem[h]
            ).wait()

        for h in range(NBUF):
            stage_and_fire(h, h)

        def round_body(r, carry):
            blk0 = r * NBUF
            for h in range(NBUF):
                wait_gather(h)
                fire_out(h, blk0 + h)
            for h in range(NBUF):
                wait_out(h)
                stage_and_fire(h, blk0 + NBUF + h)
            return carry

        lax.fori_loop(0, nround - 1, round_body, 0)

        blk0 = (nround - 1) * NBUF
        for h in range(NBUF):
            wait_gather(h)
            fire_out(h, blk0 + h)
        for h in range(NBUF):
            wait_out(h)

    return k(flat_table, idx2d)


def _mm_body(x_ref, w_ref, b_ref, o_ref):
    a = x_ref[...]
    bms = a.shape[0]
    acc = jnp.zeros((bms * 8, PROJ), jnp.float32) + b_ref[...]
    for p in range(4):
        ap = a[:, p].reshape(bms * 8, 128)
        acc = acc + jnp.dot(ap, w_ref[p], preferred_element_type=jnp.float32)
    o_ref[...] = acc


def _tc_project(cat4, proj_w3d, proj_b2d, bms):
    nslab = cat4.shape[0]
    return pl.pallas_call(
        _mm_body,
        grid=(nslab // bms,),
        in_specs=[
            pl.BlockSpec((bms, 4, 8, 128), lambda i: (i, 0, 0, 0)),
            pl.BlockSpec((4, 128, PROJ), lambda i: (0, 0, 0)),
            pl.BlockSpec((1, PROJ), lambda i: (0, 0)),
        ],
        out_specs=pl.BlockSpec((bms * 8, PROJ), lambda i: (i, 0)),
        out_shape=jax.ShapeDtypeStruct((nslab * 8, PROJ), jnp.float32),
    )(cat4, proj_w3d, proj_b2d)


def kernel(x, tables, proj_w, proj_b):
    b, l, nf = x.shape
    n = b * l
    total_rows = n * nf
    offsets = (jnp.arange(nf, dtype=jnp.int32) * VOCAB).reshape(1, 1, nf)
    idx = (
        (x.astype(jnp.int32) + offsets)
        .reshape(n // 8, 8, nf // 2, 2)
        .transpose(0, 2, 1, 3)
        .reshape(total_rows // CH, CH)
    )
    flat_table = tables.reshape(nf * VOCAB, EMB)
    cat = _sc_gather(flat_table, idx, total_rows)
    cat4 = cat.reshape(n // 8, 4, 8, 128)
    out = _tc_project(cat4, proj_w.reshape(4, 128, PROJ), proj_b.reshape(1, PROJ), 128)
    return out.reshape(b, l, PROJ)


# speedup vs baseline: 2.3554x; 2.3554x over previous
import functools

import jax
import jax.numpy as jnp
from jax import lax
from jax.experimental import pallas as pl
from jax.experimental.pallas import tpu as pltpu
from jax.experimental.pallas import tpu_sc as plsc

NUM_FIELDS = 8
VOCAB = 100000
EMB = 64
PROJ = 128

NC, NS = 2, 16
NW = NC * NS
T = 128
NBUF = 5


def _precompute_body(t_ref, w_ref, b_ref, o_ref):
    o_ref[0] = (
        jnp.dot(t_ref[0], w_ref[0], preferred_element_type=jnp.float32)
        + b_ref[...] * (1.0 / NUM_FIELDS)
    )


def _tc_precompute(tables, w3, proj_b2d, bmv):
    nf, vocab, emb = tables.shape
    return pl.pallas_call(
        _precompute_body,
        grid=(nf, vocab // bmv),
        in_specs=[
            pl.BlockSpec((1, bmv, emb), lambda i, j: (i, j, 0)),
            pl.BlockSpec((1, emb, PROJ), lambda i, j: (i, 0, 0)),
            pl.BlockSpec((1, PROJ), lambda i, j: (0, 0)),
        ],
        out_specs=pl.BlockSpec((1, bmv, PROJ), lambda i, j: (i, j, 0)),
        out_shape=jax.ShapeDtypeStruct((nf, vocab, PROJ), jnp.float32),
    )(tables, w3, proj_b2d)


def _sc_gather_add(p_tab, idx3, n_tok):
    tpw = n_tok // NW
    nblk = tpw // T
    nround = nblk // NBUF

    mesh = plsc.VectorSubcoreMesh(core_axis_name="c", subcore_axis_name="s")

    @functools.partial(
        pl.kernel,
        out_type=jax.ShapeDtypeStruct((n_tok, PROJ), jnp.float32),
        mesh=mesh,
        scratch_types=[
            pltpu.VMEM((NBUF, NUM_FIELDS, T), jnp.int32),
            pltpu.VMEM((NBUF, T, PROJ), jnp.float32),
        ]
        + [pltpu.SemaphoreType.DMA] * (3 * NBUF),
    )
    def k(p_hbm, idx_hbm, out_hbm, idx_v, acc_v, *sems):
        g0sem, gsem, osem = sems[:NBUF], sems[NBUF : 2 * NBUF], sems[2 * NBUF :]
        wid = lax.axis_index("s") * NC + lax.axis_index("c")
        blk_base = wid * nblk
        tok_base = wid * tpw

        def stage_and_fire0(h, blk_i):
            pltpu.sync_copy(idx_hbm.at[blk_base + blk_i], idx_v.at[h])
            pltpu.async_copy(
                p_hbm.at[idx_v.at[h].at[0]], acc_v.at[h], g0sem[h]
            )

        def wait0_fire_adds(h):
            pltpu.make_async_copy(
                p_hbm.at[idx_v.at[h].at[0]], acc_v.at[h], g0sem[h]
            ).wait()
            for i in range(1, NUM_FIELDS):
                pltpu.async_copy(
                    p_hbm.at[idx_v.at[h].at[i]], acc_v.at[h], gsem[h], add=True
                )

        def wait_adds_fire_out(h, blk_i):
            for i in range(1, NUM_FIELDS):
                pltpu.make_async_copy(
                    p_hbm.at[idx_v.at[h].at[i]], acc_v.at[h], gsem[h]
                ).wait()
            pltpu.async_copy(
                acc_v.at[h],
                out_hbm.at[pl.ds(tok_base + blk_i * T, T)],
                osem[h],
            )

        def wait_out(h):
            pltpu.make_async_copy(
                acc_v.at[h], out_hbm.at[pl.ds(tok_base, T)], osem[h]
            ).wait()

        for h in range(NBUF):
            stage_and_fire0(h, h)

        def round_body(r, carry):
            blk0 = r * NBUF
            for h in range(NBUF):
                wait0_fire_adds(h)
            for h in range(NBUF):
                wait_adds_fire_out(h, blk0 + h)
            for h in range(NBUF):
                wait_out(h)
                stage_and_fire0(h, blk0 + NBUF + h)
            return carry

        lax.fori_loop(0, nround - 1, round_body, 0)

        blk0 = (nround - 1) * NBUF
        for h in range(NBUF):
            wait0_fire_adds(h)
        for h in range(NBUF):
            wait_adds_fire_out(h, blk0 + h)
        for h in range(NBUF):
            wait_out(h)

    return k(p_tab, idx3)


def kernel(x, tables, proj_w, proj_b):
    b, l, nf = x.shape
    n = b * l
    offsets = (jnp.arange(nf, dtype=jnp.int32) * VOCAB).reshape(1, nf, 1)
    idx3 = (
        x.astype(jnp.int32)
        .reshape(n // T, T, nf)
        .transpose(0, 2, 1)
        + offsets
    )
    p3 = _tc_precompute(
        tables, proj_w.reshape(nf, EMB, PROJ), proj_b.reshape(1, PROJ), 2000
    )
    p_tab = p3.reshape(nf * VOCAB, PROJ)
    out2d = _sc_gather_add(p_tab, idx3, n)
    return out2d.reshape(b, l, PROJ)


# speedup vs baseline: 2.4144x; 1.0251x over previous
import functools

import jax
import jax.numpy as jnp
from jax import lax
from jax.experimental import pallas as pl
from jax.experimental.pallas import tpu as pltpu
from jax.experimental.pallas import tpu_sc as plsc

NUM_FIELDS = 8
VOCAB = 100000
EMB = 64
PROJ = 128

NC, NS = 2, 16
NW = NC * NS
T = 128
NBUF = 5


def _precompute_body(t_ref, w_ref, b_ref, o_ref):
    o_ref[0] = (
        jnp.dot(
            t_ref[0].astype(jnp.bfloat16),
            w_ref[0].astype(jnp.bfloat16),
            preferred_element_type=jnp.float32,
        )
        + b_ref[...] * (1.0 / NUM_FIELDS)
    )


def _tc_precompute(tables, w3, proj_b2d, bmv):
    nf, vocab, emb = tables.shape
    return pl.pallas_call(
        _precompute_body,
        grid=(nf, vocab // bmv),
        in_specs=[
            pl.BlockSpec((1, bmv, emb), lambda i, j: (i, j, 0)),
            pl.BlockSpec((1, emb, PROJ), lambda i, j: (i, 0, 0)),
            pl.BlockSpec((1, PROJ), lambda i, j: (0, 0)),
        ],
        out_specs=pl.BlockSpec((1, bmv, PROJ), lambda i, j: (i, j, 0)),
        out_shape=jax.ShapeDtypeStruct((nf, vocab, PROJ), jnp.float32),
    )(tables, w3, proj_b2d)


def _sc_gather_add(p_tab, idx3, n_tok):
    tpw = n_tok // NW
    nblk = tpw // T
    nround = nblk // NBUF

    mesh = plsc.VectorSubcoreMesh(core_axis_name="c", subcore_axis_name="s")

    @functools.partial(
        pl.kernel,
        out_type=jax.ShapeDtypeStruct((n_tok, PROJ), jnp.float32),
        mesh=mesh,
        scratch_types=[
            pltpu.VMEM((NBUF, NUM_FIELDS, T), jnp.int32),
            pltpu.VMEM((NBUF, T, PROJ), jnp.float32),
        ]
        + [pltpu.SemaphoreType.DMA] * (4 * NBUF),
    )
    def k(p_hbm, idx_hbm, out_hbm, idx_v, acc_v, *sems):
        isem = sems[:NBUF]
        g0sem = sems[NBUF : 2 * NBUF]
        gsem = sems[2 * NBUF : 3 * NBUF]
        osem = sems[3 * NBUF :]
        wid = lax.axis_index("s") * NC + lax.axis_index("c")
        blk_base = wid * nblk
        tok_base = wid * tpw

        def fire_idx(h, blk_i):
            pltpu.async_copy(idx_hbm.at[blk_base + blk_i], idx_v.at[h], isem[h])

        def wait_idx_fire0(h):
            pltpu.make_async_copy(
                idx_hbm.at[blk_base], idx_v.at[h], isem[h]
            ).wait()
            pltpu.async_copy(
                p_hbm.at[idx_v.at[h].at[0]], acc_v.at[h], g0sem[h]
            )

        def wait0_fire_adds(h):
            pltpu.make_async_copy(
                p_hbm.at[idx_v.at[h].at[0]], acc_v.at[h], g0sem[h]
            ).wait()
            for i in range(1, NUM_FIELDS):
                pltpu.async_copy(
                    p_hbm.at[idx_v.at[h].at[i]], acc_v.at[h], gsem[h], add=True
                )

        def wait_adds_fire_out(h, blk_i):
            for i in range(1, NUM_FIELDS):
                pltpu.make_async_copy(
                    p_hbm.at[idx_v.at[h].at[i]], acc_v.at[h], gsem[h]
                ).wait()
            pltpu.async_copy(
                acc_v.at[h],
                out_hbm.at[pl.ds(tok_base + blk_i * T, T)],
                osem[h],
            )

        def wait_out(h):
            pltpu.make_async_copy(
                acc_v.at[h], out_hbm.at[pl.ds(tok_base, T)], osem[h]
            ).wait()

        for h in range(NBUF):
            fire_idx(h, h)
        for h in range(NBUF):
            wait_idx_fire0(h)

        def round_body(r, carry):
            blk0 = r * NBUF
            for h in range(NBUF):
                wait0_fire_adds(h)
            for h in range(NBUF):
                wait_adds_fire_out(h, blk0 + h)
            for h in range(NBUF):
                wait_out(h)
                fire_idx(h, blk0 + NBUF + h)
            for h in range(NBUF):
                wait_idx_fire0(h)
            return carry

        lax.fori_loop(0, nround - 1, round_body, 0)

        for h in range(NBUF):
            wait0_fire_adds(h)
        blk0 = (nround - 1) * NBUF
        for h in range(NBUF):
            wait_adds_fire_out(h, blk0 + h)
        for h in range(NBUF):
            wait_out(h)

    return k(p_tab, idx3)


def kernel(x, tables, proj_w, proj_b):
    b, l, nf = x.shape
    n = b * l
    offsets = (jnp.arange(nf, dtype=jnp.int32) * VOCAB).reshape(1, nf, 1)
    idx3 = (
        x.astype(jnp.int32)
        .reshape(n // T, T, nf)
        .transpose(0, 2, 1)
        + offsets
    )
    p3 = _tc_precompute(
        tables, proj_w.reshape(nf, EMB, PROJ), proj_b.reshape(1, PROJ), 2000
    )
    p_tab = p3.reshape(nf * VOCAB, PROJ)
    out2d = _sc_gather_add(p_tab, idx3, n)
    return out2d.reshape(b, l, PROJ)


# speedup vs baseline: 2.7129x; 1.1237x over previous
import functools

import jax
import jax.numpy as jnp
from jax import lax
from jax.experimental import pallas as pl
from jax.experimental.pallas import tpu as pltpu
from jax.experimental.pallas import tpu_sc as plsc

NUM_FIELDS = 8
VOCAB = 100000
EMB = 64
PROJ = 128

NC, NS = 2, 16
NW = NC * NS
T = 128
NBUF = 5


def _precompute_body(t_ref, w_ref, b_ref, o_ref):
    o_ref[0] = (
        jnp.dot(
            t_ref[0].astype(jnp.bfloat16),
            w_ref[0].astype(jnp.bfloat16),
            preferred_element_type=jnp.float32,
        )
        + b_ref[...] * (1.0 / NUM_FIELDS)
    )


def _tc_precompute(tables, w3, proj_b2d, bmv):
    nf, vocab, emb = tables.shape
    return pl.pallas_call(
        _precompute_body,
        grid=(nf, vocab // bmv),
        in_specs=[
            pl.BlockSpec((1, bmv, emb), lambda i, j: (i, j, 0)),
            pl.BlockSpec((1, emb, PROJ), lambda i, j: (i, 0, 0)),
            pl.BlockSpec((1, PROJ), lambda i, j: (0, 0)),
        ],
        out_specs=pl.BlockSpec((1, bmv, PROJ), lambda i, j: (i, j, 0)),
        out_shape=jax.ShapeDtypeStruct((nf, vocab, PROJ), jnp.float32),
    )(tables, w3, proj_b2d)


def _sc_gather_add(p_tab, idx3, n_tok):
    tpw = n_tok // NW
    nblk = tpw // T
    nround = nblk // NBUF

    mesh = plsc.VectorSubcoreMesh(core_axis_name="c", subcore_axis_name="s")

    @functools.partial(
        pl.kernel,
        out_type=jax.ShapeDtypeStruct((n_tok, PROJ), jnp.float32),
        mesh=mesh,
        scratch_types=[
            pltpu.VMEM((NBUF, NUM_FIELDS, T), jnp.int32),
            pltpu.VMEM((NBUF, T, PROJ), jnp.float32),
        ]
        + [pltpu.SemaphoreType.DMA] * (4 * NBUF),
    )
    def k(p_hbm, idx_hbm, out_hbm, idx_v, acc_v, *sems):
        isem = sems[:NBUF]
        g0sem = sems[NBUF : 2 * NBUF]
        gsem = sems[2 * NBUF : 3 * NBUF]
        osem = sems[3 * NBUF :]
        wid = lax.axis_index("s") * NC + lax.axis_index("c")
        blk_base = wid * nblk
        tok_base = wid * tpw

        def fire_idx(h, blk_i):
            pltpu.async_copy(idx_hbm.at[blk_base + blk_i], idx_v.at[h], isem[h])

        def wait_idx_fire0(h):
            pltpu.make_async_copy(
                idx_hbm.at[blk_base], idx_v.at[h], isem[h]
            ).wait()
            pltpu.async_copy(
                p_hbm.at[idx_v.at[h].at[0]], acc_v.at[h], g0sem[h]
            )

        def wait0_fire_adds(h):
            pltpu.make_async_copy(
                p_hbm.at[idx_v.at[h].at[0]], acc_v.at[h], g0sem[h]
            ).wait()
            for i in range(1, NUM_FIELDS):
                pltpu.async_copy(
                    p_hbm.at[idx_v.at[h].at[i]], acc_v.at[h], gsem[h], add=True
                )

        def wait_adds_fire_out(h, blk_i):
            for i in range(1, NUM_FIELDS):
                pltpu.make_async_copy(
                    p_hbm.at[idx_v.at[h].at[i]], acc_v.at[h], gsem[h]
                ).wait()
            pltpu.async_copy(
                acc_v.at[h],
                out_hbm.at[pl.ds(tok_base + blk_i * T, T)],
                osem[h],
            )

        def wait_out(h):
            pltpu.make_async_copy(
                acc_v.at[h], out_hbm.at[pl.ds(tok_base, T)], osem[h]
            ).wait()

        for h in range(NBUF):
            fire_idx(h, h)
        for h in range(NBUF):
            wait_idx_fire0(h)

        def round_body(r, carry):
            blk0 = r * NBUF
            for h in range(NBUF):
                wait0_fire_adds(h)
            for h in range(NBUF):
                wait_adds_fire_out(h, blk0 + h)
            for h in range(NBUF):
                wait_out(h)
                fire_idx(h, blk0 + NBUF + h)
            for h in range(NBUF):
                wait_idx_fire0(h)
            return carry

        lax.fori_loop(0, nround - 1, round_body, 0)

        for h in range(NBUF):
            wait0_fire_adds(h)
        blk0 = (nround - 1) * NBUF
        for h in range(NBUF):
            wait_adds_fire_out(h, blk0 + h)
        for h in range(NBUF):
            wait_out(h)

    return k(p_tab, idx3)


def kernel(x, tables, proj_w, proj_b):
    b, l, nf = x.shape
    n = b * l
    offsets = (jnp.arange(nf, dtype=jnp.int32) * VOCAB).reshape(1, nf, 1)
    idx3 = (
        x.astype(jnp.int32)
        .reshape(n // T, T, nf)
        .transpose(0, 2, 1)
        + offsets
    )
    p3 = _tc_precompute(
        tables, proj_w.reshape(nf, EMB, PROJ), proj_b.reshape(1, PROJ), 5000
    )
    p_tab = p3.reshape(nf * VOCAB, PROJ)
    out2d = _sc_gather_add(p_tab, idx3, n)
    return out2d.reshape(b, l, PROJ)


# speedup vs baseline: 2.8020x; 1.0328x over previous
import functools

import jax
import jax.numpy as jnp
from jax import lax
from jax.experimental import pallas as pl
from jax.experimental.pallas import tpu as pltpu
from jax.experimental.pallas import tpu_sc as plsc

NUM_FIELDS = 8
VOCAB = 100000
EMB = 64
PROJ = 128

NC, NS = 2, 16
NW = NC * NS
T = 128
NBUF = 5


def _precompute_body(t_ref, w_ref, b_ref, o_ref):
    o_ref[0] = (
        jnp.dot(
            t_ref[0].astype(jnp.bfloat16),
            w_ref[0].astype(jnp.bfloat16),
            preferred_element_type=jnp.float32,
        )
        + b_ref[...] * (1.0 / NUM_FIELDS)
    )


def _tc_precompute(tables, w3, proj_b2d, bmv):
    nf, vocab, emb = tables.shape
    return pl.pallas_call(
        _precompute_body,
        grid=(nf, vocab // bmv),
        in_specs=[
            pl.BlockSpec((1, bmv, emb), lambda i, j: (i, j, 0)),
            pl.BlockSpec((1, emb, PROJ), lambda i, j: (i, 0, 0)),
            pl.BlockSpec((1, PROJ), lambda i, j: (0, 0)),
        ],
        out_specs=pl.BlockSpec((1, bmv, PROJ), lambda i, j: (i, j, 0)),
        out_shape=jax.ShapeDtypeStruct((nf, vocab, PROJ), jnp.float32),
    )(tables, w3, proj_b2d)


def _sc_gather_add(p_tab, idx3, n_tok):
    tpw = n_tok // NW
    nblk = tpw // T
    nround = nblk // NBUF

    mesh = plsc.VectorSubcoreMesh(core_axis_name="c", subcore_axis_name="s")

    @functools.partial(
        pl.kernel,
        out_type=jax.ShapeDtypeStruct((n_tok, PROJ), jnp.float32),
        mesh=mesh,
        scratch_types=[
            pltpu.VMEM((NBUF, NUM_FIELDS, T), jnp.int32),
            pltpu.VMEM((NBUF, T, PROJ), jnp.float32),
        ]
        + [pltpu.SemaphoreType.DMA] * (4 * NBUF),
    )
    def k(p_hbm, idx_hbm, out_hbm, idx_v, acc_v, *sems):
        isem = sems[:NBUF]
        g0sem = sems[NBUF : 2 * NBUF]
        gsem = sems[2 * NBUF : 3 * NBUF]
        osem = sems[3 * NBUF :]
        wid = lax.axis_index("s") * NC + lax.axis_index("c")
        blk_base = wid * nblk
        tok_base = wid * tpw

        def fire_idx(h, blk_i):
            pltpu.async_copy(idx_hbm.at[blk_base + blk_i], idx_v.at[h], isem[h])

        def wait_idx_fire0(h):
            pltpu.make_async_copy(
                idx_hbm.at[blk_base], idx_v.at[h], isem[h]
            ).wait()
            pltpu.async_copy(
                p_hbm.at[idx_v.at[h].at[0]], acc_v.at[h], g0sem[h]
            )

        def wait0_fire_adds(h):
            pltpu.make_async_copy(
                p_hbm.at[idx_v.at[h].at[0]], acc_v.at[h], g0sem[h]
            ).wait()
            for i in range(1, NUM_FIELDS):
                pltpu.async_copy(
                    p_hbm.at[idx_v.at[h].at[i]], acc_v.at[h], gsem[h], add=True
                )

        def wait_adds_fire_out(h, blk_i):
            for i in range(1, NUM_FIELDS):
                pltpu.make_async_copy(
                    p_hbm.at[idx_v.at[h].at[i]], acc_v.at[h], gsem[h]
                ).wait()
            pltpu.async_copy(
                acc_v.at[h],
                out_hbm.at[pl.ds(tok_base + blk_i * T, T)],
                osem[h],
            )

        def wait_out(h):
            pltpu.make_async_copy(
                acc_v.at[h], out_hbm.at[pl.ds(tok_base, T)], osem[h]
            ).wait()

        for h in range(NBUF):
            fire_idx(h, h)
        for h in range(NBUF):
            wait_idx_fire0(h)

        def round_body(r, carry):
            blk0 = r * NBUF
            for h in range(NBUF):
                wait0_fire_adds(h)
            for h in range(NBUF):
                wait_adds_fire_out(h, blk0 + h)
            for h in range(NBUF):
                wait_out(h)
                fire_idx(h, blk0 + NBUF + h)
            for h in range(NBUF):
                wait_idx_fire0(h)
            return carry

        lax.fori_loop(0, nround - 1, round_body, 0)

        for h in range(NBUF):
            wait0_fire_adds(h)
        blk0 = (nround - 1) * NBUF
        for h in range(NBUF):
            wait_adds_fire_out(h, blk0 + h)
        for h in range(NBUF):
            wait_out(h)

    return k(p_tab, idx3)


def kernel(x, tables, proj_w, proj_b):
    b, l, nf = x.shape
    n = b * l
    offsets = (jnp.arange(nf, dtype=jnp.int32) * VOCAB).reshape(1, nf, 1)
    idx3 = (
        x.astype(jnp.int32)
        .reshape(n // T, T, nf)
        .transpose(0, 2, 1)
        + offsets
    )
    p3 = _tc_precompute(
        tables, proj_w.reshape(nf, EMB, PROJ), proj_b.reshape(1, PROJ), 10000
    )
    p_tab = p3.reshape(nf * VOCAB, PROJ)
    out2d = _sc_gather_add(p_tab, idx3, n)
    return out2d.reshape(b, l, PROJ)


# speedup vs baseline: 2.8105x; 1.0030x over previous
import functools

import jax
import jax.numpy as jnp
from jax import lax
from jax.experimental import pallas as pl
from jax.experimental.pallas import tpu as pltpu
from jax.experimental.pallas import tpu_sc as plsc

NUM_FIELDS = 8
VOCAB = 100000
EMB = 64
PROJ = 128

NC, NS = 2, 16
NW = NC * NS
T = 128
NBUF = 5


def _precompute_body(t_ref, w_ref, b_ref, o_ref):
    o_ref[0] = (
        jnp.dot(
            t_ref[0].astype(jnp.bfloat16),
            w_ref[0].astype(jnp.bfloat16),
            preferred_element_type=jnp.float32,
        )
        + b_ref[...] * (1.0 / NUM_FIELDS)
    )


def _tc_precompute(tables, w3, proj_b2d, bmv):
    nf, vocab, emb = tables.shape
    return pl.pallas_call(
        _precompute_body,
        grid=(nf, vocab // bmv),
        in_specs=[
            pl.BlockSpec((1, bmv, emb), lambda i, j: (i, j, 0)),
            pl.BlockSpec((1, emb, PROJ), lambda i, j: (i, 0, 0)),
            pl.BlockSpec((1, PROJ), lambda i, j: (0, 0)),
        ],
        out_specs=pl.BlockSpec((1, bmv, PROJ), lambda i, j: (i, j, 0)),
        out_shape=jax.ShapeDtypeStruct((nf, vocab, PROJ), jnp.float32),
    )(tables, w3, proj_b2d)


def _sc_gather_add(p_tab, idx3, n_tok):
    tpw = n_tok // NW
    nblk = tpw // T
    nround = nblk // NBUF

    mesh = plsc.VectorSubcoreMesh(core_axis_name="c", subcore_axis_name="s")

    @functools.partial(
        pl.kernel,
        out_type=jax.ShapeDtypeStruct((n_tok, PROJ), jnp.float32),
        mesh=mesh,
        scratch_types=[
            pltpu.VMEM((NBUF, NUM_FIELDS, T), jnp.int32),
            pltpu.VMEM((NBUF, T, PROJ), jnp.float32),
        ]
        + [pltpu.SemaphoreType.DMA] * (4 * NBUF),
    )
    def k(p_hbm, idx_hbm, out_hbm, idx_v, acc_v, *sems):
        isem = sems[:NBUF]
        g0sem = sems[NBUF : 2 * NBUF]
        gsem = sems[2 * NBUF : 3 * NBUF]
        osem = sems[3 * NBUF :]
        wid = lax.axis_index("s") * NC + lax.axis_index("c")
        blk_base = wid * nblk
        tok_base = wid * tpw

        def fire_idx(h, blk_i):
            pltpu.async_copy(idx_hbm.at[blk_base + blk_i], idx_v.at[h], isem[h])

        def wait_idx_fire0(h):
            pltpu.make_async_copy(
                idx_hbm.at[blk_base], idx_v.at[h], isem[h]
            ).wait()
            pltpu.async_copy(
                p_hbm.at[idx_v.at[h].at[0]], acc_v.at[h], g0sem[h]
            )

        def wait0_fire_adds(h):
            pltpu.make_async_copy(
                p_hbm.at[idx_v.at[h].at[0]], acc_v.at[h], g0sem[h]
            ).wait()
            for i in range(1, NUM_FIELDS):
                pltpu.async_copy(
                    p_hbm.at[idx_v.at[h].at[i]], acc_v.at[h], gsem[h], add=True
                )

        def wait_adds_fire_out(h, blk_i):
            for i in range(1, NUM_FIELDS):
                pltpu.make_async_copy(
                    p_hbm.at[idx_v.at[h].at[i]], acc_v.at[h], gsem[h]
                ).wait()
            pltpu.async_copy(
                acc_v.at[h],
                out_hbm.at[pl.ds(tok_base + blk_i * T, T)],
                osem[h],
            )

        def wait_out(h):
            pltpu.make_async_copy(
                acc_v.at[h], out_hbm.at[pl.ds(tok_base, T)], osem[h]
            ).wait()

        for h in range(NBUF):
            fire_idx(h, h)
        for h in range(NBUF):
            wait_idx_fire0(h)

        def round_body(r, carry):
            blk0 = r * NBUF
            for h in range(NBUF):
                wait0_fire_adds(h)
            for h in range(NBUF):
                wait_adds_fire_out(h, blk0 + h)
            for h in range(NBUF):
                wait_out(h)
                fire_idx(h, blk0 + NBUF + h)
            for h in range(NBUF):
                wait_idx_fire0(h)
            return carry

        lax.fori_loop(0, nround - 1, round_body, 0)

        for h in range(NBUF):
            wait0_fire_adds(h)
        blk0 = (nround - 1) * NBUF
        for h in range(NBUF):
            wait_adds_fire_out(h, blk0 + h)
        for h in range(NBUF):
            wait_out(h)

    return k(p_tab, idx3)


def kernel(x, tables, proj_w, proj_b):
    b, l, nf = x.shape
    n = b * l
    offsets = (jnp.arange(nf, dtype=jnp.int32) * VOCAB).reshape(1, nf, 1)
    idx3 = (
        x.astype(jnp.int32)
        .reshape(n // T, T, nf)
        .transpose(0, 2, 1)
        + offsets
    )
    p3 = _tc_precompute(
        tables, proj_w.reshape(nf, EMB, PROJ), proj_b.reshape(1, PROJ), 20000
    )
    p_tab = p3.reshape(nf * VOCAB, PROJ)
    out2d = _sc_gather_add(p_tab, idx3, n)
    return out2d.reshape(b, l, PROJ)
